# baseline (speedup 1.0000x reference)
import jax
import jax.numpy as jnp
from jax import lax
from jax.experimental import pallas as pl
from jax.experimental.pallas import tpu as pltpu
from jax.experimental.pallas import tpu_sc as plsc

Q = 1024
D = 128
K = 100000
KT = 4096
NTILES = 25
KPAD = NTILES * KT
G1 = 128
G2 = 1024
NB1 = KPAD // G1
NB1P = 896
NB2 = KPAD // G2
NB2P = 128
BT = KT // G1
NEG = -3.0e38

NC = 2
NS = 16
L = 16
NW = NC * NS
QW = Q // NW


def _phase1_body(q_ref, k_ref, t_ref, s_ref, m1_ref):
    j = pl.program_id(0)
    q2 = q_ref[...] * 2.0

    def work(kt_ref):
        rows = []
        for sub in range(KT // 256):
            kk = kt_ref[pl.ds(sub * 256, 256), :]
            dot2 = lax.dot_general(
                q2, kk, (((1,), (1,)), ((), ())),
                preferred_element_type=jnp.float32,
                precision=lax.Precision.DEFAULT,
            )
            ksq = jnp.sum(kk * kk, axis=1)[None, :]
            s = dot2 - ksq
            for h in range(2):
                lb = sub * 2 + h
                sh = s[:, h * G1:(h + 1) * G1]
                s_ref[lb] = sh
                rows.append(jnp.max(sh, axis=1, keepdims=True))
        m1t = jnp.concatenate(rows, axis=1)
        for c in range(128 // BT):
            @pl.when(j % (128 // BT) == c)
            def _(c=c):
                m1_ref[:, c * BT:(c + 1) * BT] = m1t

    @pl.when(j < NTILES - 1)
    def _():
        work(k_ref)

    @pl.when(j == NTILES - 1)
    def _():
        work(t_ref)


def _phase1(queries, keys, tail):
    qn = queries.shape[0]
    return pl.pallas_call(
        _phase1_body,
        grid=(NTILES,),
        in_specs=[
            pl.BlockSpec((qn, D), lambda j: (0, 0)),
            pl.BlockSpec((KT, D), lambda j: (jnp.minimum(j, NTILES - 2), 0)),
            pl.BlockSpec((KT, D), lambda j: (0, 0)),
        ],
        out_specs=[
            pl.BlockSpec((BT, qn, G1), lambda j: (j, 0, 0)),
            pl.BlockSpec((qn, 128), lambda j: (0, j // (128 // BT))),
        ],
        out_shape=[
            jax.ShapeDtypeStruct((NB1, qn, G1), jnp.float32),
            jax.ShapeDtypeStruct((qn, NB1P), jnp.float32),
        ],
    )(queries, keys, tail)


_GATHER_DNUMS = lax.GatherDimensionNumbers(
    offset_dims=(), collapsed_slice_dims=(0,), start_index_map=(0,))


def _permute(x, perm):
    return lax.gather(x, perm[:, None], _GATHER_DNUMS, slice_sizes=(1,),
                      mode=lax.GatherScatterMode.PROMISE_IN_BOUNDS)


def _merge16(av, ai, bv, bi):
    rbv = lax.rev(bv, (0,))
    rbi = lax.rev(bi, (0,))
    take = (av > rbv) | ((av == rbv) & (ai < rbi))
    cv = jnp.where(take, av, rbv)
    ci = jnp.where(take, ai, rbi)
    cv, ci = plsc.sort_key_val(cv, ci, descending=True)
    return cv, ci


def _merge_tree(pairs):
    while len(pairs) > 1:
        nxt = []
        for a in range(0, len(pairs) - 1, 2):
            (av, ai), (bv, bi) = pairs[a], pairs[a + 1]
            nxt.append(_merge16(av, ai, bv, bi))
        if len(pairs) % 2:
            nxt.append(pairs[-1])
        pairs = nxt
    return pairs[0]


def _tie_fix(v, i):
    lane = lax.iota(jnp.int32, L)
    for parity in (0, 1, 0, 1):
        partner = lane - 2 * ((lane - parity) % 2) + 1
        partner = jnp.clip(partner, 0, L - 1)
        pv = _permute(v, partner)
        pi = _permute(i, partner)
        win = (v > pv) | ((v == pv) & (i < pi)) | (partner == lane)
        first = partner > lane
        keep_self = jnp.where(first, win, ~win)
        v = jnp.where(keep_self, v, pv)
        i = jnp.where(keep_self, i, pi)
    return v, i


def _topk_sc_body(qn, m1_hbm, sc_hbm, vals_hbm, idx_hbm,
                  m1_v, gidx_v, gath_v, ov_v, oi_v, sem):
    qw = qn // NW
    wid = lax.axis_index("c") * NS + lax.axis_index("s")
    base = wid * qw
    pltpu.sync_copy(m1_hbm.at[pl.ds(base * NB1P, qw * NB1P)], m1_v)

    lane = lax.iota(jnp.int32, L)

    def one_query(r, _):
        q = base + r
        pairs = []
        for c in range(NB1 // L):
            v = m1_v[pl.ds(r * NB1P + c * L, L)]
            i = lane + c * L
            pairs.append(plsc.sort_key_val(v, i, descending=True))
        bv, bi = _merge_tree(pairs)

        gidx_v[...] = bi * qn + q
        pltpu.async_copy(sc_hbm.at[gidx_v], gath_v, sem).wait()

        pairs = []
        for blk in range(L):
            b = _permute(bi, jnp.full((L,), blk, jnp.int32))
            for seg in range(G1 // L):
                v = gath_v[blk, pl.ds(seg * L, L)]
                i = b * G1 + (seg * L) + lane
                pairs.append(plsc.sort_key_val(v, i, descending=True))
        fv, fi = _merge_tree(pairs)
        fv, fi = _tie_fix(fv, fi)

        ov_v[pl.ds(r * L, L)] = fv
        oi_v[pl.ds(r * L, L)] = fi
        return 0

    lax.fori_loop(0, qw, one_query, 0)
    pltpu.sync_copy(ov_v, vals_hbm.at[pl.ds(base * L, qw * L)])
    pltpu.sync_copy(oi_v, idx_hbm.at[pl.ds(base * L, qw * L)])


def _topk_sc(m1_flat, scores_rows, qn):
    import functools
    qw = qn // NW
    kern = pl.kernel(
        functools.partial(_topk_sc_body, qn),
        out_type=[
            jax.ShapeDtypeStruct((qn * L,), jnp.float32),
            jax.ShapeDtypeStruct((qn * L,), jnp.int32),
        ],
        mesh=plsc.VectorSubcoreMesh(core_axis_name="c", subcore_axis_name="s"),
        compiler_params=pltpu.CompilerParams(needs_layout_passes=False),
        scratch_types=[
            pltpu.VMEM((qw * NB1P,), jnp.float32),
            pltpu.VMEM((L,), jnp.int32),
            pltpu.VMEM((L, G1), jnp.float32),
            pltpu.VMEM((qw * L,), jnp.float32),
            pltpu.VMEM((qw * L,), jnp.int32),
            pltpu.SemaphoreType.DMA,
        ],
    )
    return kern(m1_flat, scores_rows)


def kernel(queries, keys, k):
    pad = jnp.zeros((KPAD - K, D), keys.dtype).at[:, 0].set(1e18)
    tail = jnp.concatenate([keys[(NTILES - 1) * KT:], pad], axis=0)
    nsplit = 2
    qh = Q // nsplit
    vparts, iparts = [], []
    for part in range(nsplit):
        qs = queries[part * qh:(part + 1) * qh]
        scores, m1q = _phase1(qs, keys, tail)
        vf, i_f = _topk_sc(
            m1q.reshape(-1), scores.reshape(NB1 * qh, G1), qh)
        vparts.append(vf.reshape(qh, L))
        iparts.append(i_f.reshape(qh, L))
    qsq = jnp.sum(queries * queries, axis=1, keepdims=True)
    vals = jnp.concatenate(vparts, axis=0) - qsq
    idx = jnp.concatenate(iparts, axis=0)
    idx = idx + jnp.asarray(k - k, idx.dtype)
    return vals, idx


# speedup vs baseline: 1.1270x; 1.1270x over previous
import jax
import jax.numpy as jnp
from jax import lax
from jax.experimental import pallas as pl
from jax.experimental.pallas import tpu as pltpu
from jax.experimental.pallas import tpu_sc as plsc

Q = 1024
D = 128
K = 100000
KT = 4096
NTILES = 25
KPAD = NTILES * KT
G1 = 128
G2 = 1024
NB1 = KPAD // G1
NB1P = 896
NB2 = KPAD // G2
NB2P = 128
BT = KT // G1
NEG = -3.0e38

NC = 2
NS = 16
L = 16
NW = NC * NS
QW = Q // NW


def _phase1_body(q_ref, k_ref, t_ref, s_ref, m1_ref):
    j = pl.program_id(0)
    q2 = q_ref[...] * 2.0

    def work(kt_ref):
        rows = []
        for sub in range(KT // 256):
            kk = kt_ref[pl.ds(sub * 256, 256), :]
            dot2 = lax.dot_general(
                q2, kk, (((1,), (1,)), ((), ())),
                preferred_element_type=jnp.float32,
                precision=lax.Precision.DEFAULT,
            )
            ksq = jnp.sum(kk * kk, axis=1)[None, :]
            s = dot2 - ksq
            for h in range(2):
                lb = sub * 2 + h
                sh = s[:, h * G1:(h + 1) * G1]
                s_ref[lb] = sh
                rows.append(jnp.max(sh, axis=1, keepdims=True))
        m1t = jnp.concatenate(rows, axis=1)
        for c in range(128 // BT):
            @pl.when(j % (128 // BT) == c)
            def _(c=c):
                m1_ref[:, c * BT:(c + 1) * BT] = m1t

    @pl.when(j < NTILES - 1)
    def _():
        work(k_ref)

    @pl.when(j == NTILES - 1)
    def _():
        work(t_ref)


def _phase1(queries, keys, tail):
    qn = queries.shape[0]
    return pl.pallas_call(
        _phase1_body,
        grid=(NTILES,),
        in_specs=[
            pl.BlockSpec((qn, D), lambda j: (0, 0)),
            pl.BlockSpec((KT, D), lambda j: (jnp.minimum(j, NTILES - 2), 0)),
            pl.BlockSpec((KT, D), lambda j: (0, 0)),
        ],
        out_specs=[
            pl.BlockSpec((BT, qn, G1), lambda j: (j, 0, 0)),
            pl.BlockSpec((qn, 128), lambda j: (0, j // (128 // BT))),
        ],
        out_shape=[
            jax.ShapeDtypeStruct((NB1, qn, G1), jnp.float32),
            jax.ShapeDtypeStruct((qn, NB1P), jnp.float32),
        ],
    )(queries, keys, tail)


_GATHER_DNUMS = lax.GatherDimensionNumbers(
    offset_dims=(), collapsed_slice_dims=(0,), start_index_map=(0,))


def _permute(x, perm):
    return lax.gather(x, perm[:, None], _GATHER_DNUMS, slice_sizes=(1,),
                      mode=lax.GatherScatterMode.PROMISE_IN_BOUNDS)


def _merge16(av, ai, bv, bi):
    rbv = lax.rev(bv, (0,))
    rbi = lax.rev(bi, (0,))
    take = (av > rbv) | ((av == rbv) & (ai < rbi))
    cv = jnp.where(take, av, rbv)
    ci = jnp.where(take, ai, rbi)
    cv, ci = plsc.sort_key_val(cv, ci, descending=True)
    return cv, ci


def _merge_tree(pairs):
    while len(pairs) > 1:
        nxt = []
        for a in range(0, len(pairs) - 1, 2):
            (av, ai), (bv, bi) = pairs[a], pairs[a + 1]
            nxt.append(_merge16(av, ai, bv, bi))
        if len(pairs) % 2:
            nxt.append(pairs[-1])
        pairs = nxt
    return pairs[0]


def _tie_fix(v, i):
    lane = lax.iota(jnp.int32, L)
    for parity in (0, 1, 0, 1):
        partner = lane - 2 * ((lane - parity) % 2) + 1
        partner = jnp.clip(partner, 0, L - 1)
        pv = _permute(v, partner)
        pi = _permute(i, partner)
        win = (v > pv) | ((v == pv) & (i < pi)) | (partner == lane)
        first = partner > lane
        keep_self = jnp.where(first, win, ~win)
        v = jnp.where(keep_self, v, pv)
        i = jnp.where(keep_self, i, pi)
    return v, i


def _topk_sc_body(qn, m1_hbm, sc_hbm, vals_hbm, idx_hbm,
                  m1_v, gidx_v, gath_v, ov_v, oi_v, sem):
    qw = qn // NW
    wid = lax.axis_index("c") * NS + lax.axis_index("s")
    base = wid * qw
    pltpu.sync_copy(m1_hbm.at[pl.ds(base * NB1P, qw * NB1P)], m1_v)

    lane = lax.iota(jnp.int32, L)

    def phase2(r):
        pairs = []
        for c in range(NB1 // L):
            v = m1_v[pl.ds(r * NB1P + c * L, L)]
            i = lane + c * L
            pairs.append(plsc.sort_key_val(v, i, descending=True))
        return _merge_tree(pairs)

    def one_query(r, carry):
        q = base + r
        bv, bi = carry
        gidx_v[...] = bi * qn + q
        cp = pltpu.async_copy(sc_hbm.at[gidx_v], gath_v, sem)
        nbv, nbi = phase2(jnp.minimum(r + 1, qw - 1))
        cp.wait()

        pairs = []
        for blk in range(L):
            b = _permute(bi, jnp.full((L,), blk, jnp.int32))
            for seg in range(G1 // L):
                v = gath_v[blk, pl.ds(seg * L, L)]
                i = b * G1 + (seg * L) + lane
                pairs.append(plsc.sort_key_val(v, i, descending=True))
        fv, fi = _merge_tree(pairs)
        fv, fi = _tie_fix(fv, fi)

        ov_v[pl.ds(r * L, L)] = fv
        oi_v[pl.ds(r * L, L)] = fi
        return nbv, nbi

    lax.fori_loop(0, qw, one_query, phase2(0))
    pltpu.sync_copy(ov_v, vals_hbm.at[pl.ds(base * L, qw * L)])
    pltpu.sync_copy(oi_v, idx_hbm.at[pl.ds(base * L, qw * L)])


def _topk_sc(m1_flat, scores_rows, qn):
    import functools
    qw = qn // NW
    kern = pl.kernel(
        functools.partial(_topk_sc_body, qn),
        out_type=[
            jax.ShapeDtypeStruct((qn * L,), jnp.float32),
            jax.ShapeDtypeStruct((qn * L,), jnp.int32),
        ],
        mesh=plsc.VectorSubcoreMesh(core_axis_name="c", subcore_axis_name="s"),
        compiler_params=pltpu.CompilerParams(needs_layout_passes=False),
        scratch_types=[
            pltpu.VMEM((qw * NB1P,), jnp.float32),
            pltpu.VMEM((L,), jnp.int32),
            pltpu.VMEM((L, G1), jnp.float32),
            pltpu.VMEM((qw * L,), jnp.float32),
            pltpu.VMEM((qw * L,), jnp.int32),
            pltpu.SemaphoreType.DMA,
        ],
    )
    return kern(m1_flat, scores_rows)


def kernel(queries, keys, k):
    pad = jnp.zeros((KPAD - K, D), keys.dtype).at[:, 0].set(1e18)
    tail = jnp.concatenate([keys[(NTILES - 1) * KT:], pad], axis=0)
    nsplit = 1
    qh = Q // nsplit
    vparts, iparts = [], []
    for part in range(nsplit):
        qs = queries[part * qh:(part + 1) * qh]
        scores, m1q = _phase1(qs, keys, tail)
        vf, i_f = _topk_sc(
            m1q.reshape(-1), scores.reshape(NB1 * qh, G1), qh)
        vparts.append(vf.reshape(qh, L))
        iparts.append(i_f.reshape(qh, L))
    qsq = jnp.sum(queries * queries, axis=1, keepdims=True)
    vals = jnp.concatenate(vparts, axis=0) - qsq
    idx = jnp.concatenate(iparts, axis=0)
    idx = idx + jnp.asarray(k - k, idx.dtype)
    return vals, idx


# speedup vs baseline: 1.2645x; 1.1221x over previous
import jax
import jax.numpy as jnp
from jax import lax
from jax.experimental import pallas as pl
from jax.experimental.pallas import tpu as pltpu
from jax.experimental.pallas import tpu_sc as plsc

Q = 1024
D = 128
K = 100000
KT = 4096
NTILES = 25
KPAD = NTILES * KT
G1 = 128
G2 = 1024
NB1 = KPAD // G1
NB1P = 896
NB2 = KPAD // G2
NB2P = 128
BT = KT // G1
NEG = -3.0e38

NC = 2
NS = 16
L = 16
NW = NC * NS
QW = Q // NW


def _phase1_body(q_ref, k_ref, t_ref, s_ref, m1_ref):
    j = pl.program_id(0)
    q2 = q_ref[...] * 2.0
    ones_row = jnp.ones((1, D), jnp.float32)

    def work(kt_ref):
        rows = []
        for sub in range(KT // 256):
            kk = kt_ref[pl.ds(sub * 256, 256), :]
            dot2 = lax.dot_general(
                q2, kk, (((1,), (1,)), ((), ())),
                preferred_element_type=jnp.float32,
                precision=lax.Precision.DEFAULT,
            )
            ksq = lax.dot_general(
                ones_row, kk * kk, (((1,), (1,)), ((), ())),
                preferred_element_type=jnp.float32,
                precision=lax.Precision.HIGHEST,
            )
            s = dot2 - ksq
            for h in range(2):
                lb = sub * 2 + h
                sh = s[:, h * G1:(h + 1) * G1]
                s_ref[lb] = sh
                rows.append(jnp.max(sh, axis=1, keepdims=True))
        m1t = jnp.concatenate(rows, axis=1)
        for c in range(128 // BT):
            @pl.when(j % (128 // BT) == c)
            def _(c=c):
                m1_ref[:, c * BT:(c + 1) * BT] = m1t

    @pl.when(j < NTILES - 1)
    def _():
        work(k_ref)

    @pl.when(j == NTILES - 1)
    def _():
        work(t_ref)


def _phase1(queries, keys, tail):
    qn = queries.shape[0]
    return pl.pallas_call(
        _phase1_body,
        grid=(NTILES,),
        in_specs=[
            pl.BlockSpec((qn, D), lambda j: (0, 0)),
            pl.BlockSpec((KT, D), lambda j: (jnp.minimum(j, NTILES - 2), 0)),
            pl.BlockSpec((KT, D), lambda j: (0, 0)),
        ],
        out_specs=[
            pl.BlockSpec((BT, qn, G1), lambda j: (j, 0, 0)),
            pl.BlockSpec((qn, 128), lambda j: (0, j // (128 // BT))),
        ],
        out_shape=[
            jax.ShapeDtypeStruct((NB1, qn, G1), jnp.float32),
            jax.ShapeDtypeStruct((qn, NB1P), jnp.float32),
        ],
    )(queries, keys, tail)


_GATHER_DNUMS = lax.GatherDimensionNumbers(
    offset_dims=(), collapsed_slice_dims=(0,), start_index_map=(0,))


def _permute(x, perm):
    return lax.gather(x, perm[:, None], _GATHER_DNUMS, slice_sizes=(1,),
                      mode=lax.GatherScatterMode.PROMISE_IN_BOUNDS)


def _merge16(av, ai, bv, bi):
    rbv = lax.rev(bv, (0,))
    rbi = lax.rev(bi, (0,))
    take = (av > rbv) | ((av == rbv) & (ai < rbi))
    cv = jnp.where(take, av, rbv)
    ci = jnp.where(take, ai, rbi)
    cv, ci = plsc.sort_key_val(cv, ci, descending=True)
    return cv, ci


def _merge_tree(pairs):
    while len(pairs) > 1:
        nxt = []
        for a in range(0, len(pairs) - 1, 2):
            (av, ai), (bv, bi) = pairs[a], pairs[a + 1]
            nxt.append(_merge16(av, ai, bv, bi))
        if len(pairs) % 2:
            nxt.append(pairs[-1])
        pairs = nxt
    return pairs[0]


def _tie_fix(v, i):
    lane = lax.iota(jnp.int32, L)
    for parity in (0, 1, 0, 1):
        partner = lane - 2 * ((lane - parity) % 2) + 1
        partner = jnp.clip(partner, 0, L - 1)
        pv = _permute(v, partner)
        pi = _permute(i, partner)
        win = (v > pv) | ((v == pv) & (i < pi)) | (partner == lane)
        first = partner > lane
        keep_self = jnp.where(first, win, ~win)
        v = jnp.where(keep_self, v, pv)
        i = jnp.where(keep_self, i, pi)
    return v, i


def _topk_sc_body(qn, m1_hbm, sc_hbm, vals_hbm, idx_hbm,
                  m1_v, gidx_v, gath_v, ov_v, oi_v, sem):
    qw = qn // NW
    wid = lax.axis_index("c") * NS + lax.axis_index("s")
    base = wid * qw
    pltpu.sync_copy(m1_hbm.at[pl.ds(base * NB1P, qw * NB1P)], m1_v)

    lane = lax.iota(jnp.int32, L)

    def phase2(r):
        pairs = []
        for c in range(NB1 // L):
            v = m1_v[pl.ds(r * NB1P + c * L, L)]
            i = lane + c * L
            pairs.append(plsc.sort_key_val(v, i, descending=True))
        return _merge_tree(pairs)

    def one_query(r, carry):
        q = base + r
        bv, bi = carry
        gidx_v[...] = bi * qn + q
        cp = pltpu.async_copy(sc_hbm.at[gidx_v], gath_v, sem)
        nbv, nbi = phase2(jnp.minimum(r + 1, qw - 1))
        cp.wait()

        pairs = []
        for blk in range(L):
            b = _permute(bi, jnp.full((L,), blk, jnp.int32))
            for seg in range(G1 // L):
                v = gath_v[blk, pl.ds(seg * L, L)]
                i = b * G1 + (seg * L) + lane
                pairs.append(plsc.sort_key_val(v, i, descending=True))
        fv, fi = _merge_tree(pairs)
        fv, fi = _tie_fix(fv, fi)

        ov_v[pl.ds(r * L, L)] = fv
        oi_v[pl.ds(r * L, L)] = fi
        return nbv, nbi

    lax.fori_loop(0, qw, one_query, phase2(0))
    pltpu.sync_copy(ov_v, vals_hbm.at[pl.ds(base * L, qw * L)])
    pltpu.sync_copy(oi_v, idx_hbm.at[pl.ds(base * L, qw * L)])


def _topk_sc(m1_flat, scores_rows, qn):
    import functools
    qw = qn // NW
    kern = pl.kernel(
        functools.partial(_topk_sc_body, qn),
        out_type=[
            jax.ShapeDtypeStruct((qn * L,), jnp.float32),
            jax.ShapeDtypeStruct((qn * L,), jnp.int32),
        ],
        mesh=plsc.VectorSubcoreMesh(core_axis_name="c", subcore_axis_name="s"),
        compiler_params=pltpu.CompilerParams(needs_layout_passes=False),
        scratch_types=[
            pltpu.VMEM((qw * NB1P,), jnp.float32),
            pltpu.VMEM((L,), jnp.int32),
            pltpu.VMEM((L, G1), jnp.float32),
            pltpu.VMEM((qw * L,), jnp.float32),
            pltpu.VMEM((qw * L,), jnp.int32),
            pltpu.SemaphoreType.DMA,
        ],
    )
    return kern(m1_flat, scores_rows)


def kernel(queries, keys, k):
    pad = jnp.zeros((KPAD - K, D), keys.dtype).at[:, 0].set(1e18)
    tail = jnp.concatenate([keys[(NTILES - 1) * KT:], pad], axis=0)
    nsplit = 1
    qh = Q // nsplit
    vparts, iparts = [], []
    for part in range(nsplit):
        qs = queries[part * qh:(part + 1) * qh]
        scores, m1q = _phase1(qs, keys, tail)
        vf, i_f = _topk_sc(
            m1q.reshape(-1), scores.reshape(NB1 * qh, G1), qh)
        vparts.append(vf.reshape(qh, L))
        iparts.append(i_f.reshape(qh, L))
    qsq = jnp.sum(queries * queries, axis=1, keepdims=True)
    vals = jnp.concatenate(vparts, axis=0) - qsq
    idx = jnp.concatenate(iparts, axis=0)
    idx = idx + jnp.asarray(k - k, idx.dtype)
    return vals, idx


# speedup vs baseline: 1.3008x; 1.0287x over previous
import jax
import jax.numpy as jnp
from jax import lax
from jax.experimental import pallas as pl
from jax.experimental.pallas import tpu as pltpu
from jax.experimental.pallas import tpu_sc as plsc

Q = 1024
D = 128
K = 100000
KT = 4096
NTILES = 25
KPAD = NTILES * KT
G1 = 128
G2 = 1024
NB1 = KPAD // G1
NB1P = 896
NB2 = KPAD // G2
NB2P = 128
BT = KT // G1
NEG = -3.0e38

NC = 2
NS = 16
L = 16
NW = NC * NS
QW = Q // NW


def _phase1_body(q_ref, k_ref, t_ref, s_ref, m1_ref):
    j = pl.program_id(0)
    q2 = q_ref[...] * 2.0
    ones_row = jnp.ones((1, D), jnp.float32)

    def work(kt_ref):
        rows = []
        for sub in range(KT // 256):
            kk = kt_ref[pl.ds(sub * 256, 256), :]
            dot2 = lax.dot_general(
                q2, kk, (((1,), (1,)), ((), ())),
                preferred_element_type=jnp.float32,
                precision=lax.Precision.DEFAULT,
            )
            ksq = lax.dot_general(
                ones_row, kk * kk, (((1,), (1,)), ((), ())),
                preferred_element_type=jnp.float32,
                precision=lax.Precision.HIGHEST,
            )
            s = dot2 - ksq
            for h in range(2):
                lb = sub * 2 + h
                sh = s[:, h * G1:(h + 1) * G1]
                s_ref[lb] = sh
                rows.append(jnp.max(sh, axis=1, keepdims=True))
        m1t = jnp.concatenate(rows, axis=1)
        for c in range(128 // BT):
            @pl.when(j % (128 // BT) == c)
            def _(c=c):
                m1_ref[:, c * BT:(c + 1) * BT] = m1t

    @pl.when(j < NTILES - 1)
    def _():
        work(k_ref)

    @pl.when(j == NTILES - 1)
    def _():
        work(t_ref)


def _phase1(queries, keys, tail):
    qn = queries.shape[0]
    return pl.pallas_call(
        _phase1_body,
        grid=(NTILES,),
        in_specs=[
            pl.BlockSpec((qn, D), lambda j: (0, 0)),
            pl.BlockSpec((KT, D), lambda j: (jnp.minimum(j, NTILES - 2), 0)),
            pl.BlockSpec((KT, D), lambda j: (0, 0)),
        ],
        out_specs=[
            pl.BlockSpec((BT, qn, G1), lambda j: (j, 0, 0)),
            pl.BlockSpec((qn, 128), lambda j: (0, j // (128 // BT))),
        ],
        out_shape=[
            jax.ShapeDtypeStruct((NB1, qn, G1), jnp.float32),
            jax.ShapeDtypeStruct((qn, NB1P), jnp.float32),
        ],
    )(queries, keys, tail)


_GATHER_DNUMS = lax.GatherDimensionNumbers(
    offset_dims=(), collapsed_slice_dims=(0,), start_index_map=(0,))


def _permute(x, perm):
    return lax.gather(x, perm[:, None], _GATHER_DNUMS, slice_sizes=(1,),
                      mode=lax.GatherScatterMode.PROMISE_IN_BOUNDS)


def _merge16(av, ai, bv, bi):
    rbv = lax.rev(bv, (0,))
    rbi = lax.rev(bi, (0,))
    take = (av > rbv) | ((av == rbv) & (ai < rbi))
    cv = jnp.where(take, av, rbv)
    ci = jnp.where(take, ai, rbi)
    cv, ci = plsc.sort_key_val(cv, ci, descending=True)
    return cv, ci


def _merge_tree(pairs):
    while len(pairs) > 1:
        nxt = []
        for a in range(0, len(pairs) - 1, 2):
            (av, ai), (bv, bi) = pairs[a], pairs[a + 1]
            nxt.append(_merge16(av, ai, bv, bi))
        if len(pairs) % 2:
            nxt.append(pairs[-1])
        pairs = nxt
    return pairs[0]


def _tie_fix(v, i):
    lane = lax.iota(jnp.int32, L)
    for parity in (0, 1, 0, 1):
        partner = lane - 2 * ((lane - parity) % 2) + 1
        partner = jnp.clip(partner, 0, L - 1)
        pv = _permute(v, partner)
        pi = _permute(i, partner)
        win = (v > pv) | ((v == pv) & (i < pi)) | (partner == lane)
        first = partner > lane
        keep_self = jnp.where(first, win, ~win)
        v = jnp.where(keep_self, v, pv)
        i = jnp.where(keep_self, i, pi)
    return v, i


def _topk_sc_body(qn, m1_hbm, sc_hbm, vals_hbm, idx_hbm,
                  m1_v, gidx_v, gath_v, ov_v, oi_v, sem):
    qw = qn // NW
    wid = lax.axis_index("c") * NS + lax.axis_index("s")
    base = wid * qw
    pltpu.sync_copy(m1_hbm.at[pl.ds(base * NB1P, qw * NB1P)], m1_v)

    lane = lax.iota(jnp.int32, L)

    def phase2(r):
        pairs = []
        for c in range(NB1 // L):
            v = m1_v[pl.ds(r * NB1P + c * L, L)]
            i = lane + c * L
            pairs.append(plsc.sort_key_val(v, i, descending=True))
        return _merge_tree(pairs)

    def one_query(r, carry):
        q = base + r
        bv, bi = carry
        gidx_v[...] = bi * qn + q
        cp = pltpu.async_copy(sc_hbm.at[gidx_v], gath_v, sem)
        nbv, nbi = phase2(jnp.minimum(r + 1, qw - 1))
        cp.wait()

        nseg = L * G1 // L // 1
        pairs = []
        for g in range(nseg // L):
            segid = lane + g * L
            rowv = segid >> 3
            colbase = (segid & 7) * L
            m = plsc.load_gather(gath_v, [rowv, colbase])
            for off in range(1, L):
                m = jnp.maximum(
                    m, plsc.load_gather(gath_v, [rowv, colbase + off]))
            pairs.append(plsc.sort_key_val(m, segid, descending=True))
        sgv, sgi = _merge_tree(pairs)

        pairs = []
        for mth in range(L):
            s = _permute(sgi, jnp.full((L,), mth, jnp.int32))
            blkv = s >> 3
            b = _permute(bi, blkv)
            v = plsc.load_gather(gath_v, [blkv, (s & 7) * L + lane])
            i = b * G1 + (s & 7) * L + lane
            pairs.append(plsc.sort_key_val(v, i, descending=True))
        fv, fi = _merge_tree(pairs)
        fv, fi = _tie_fix(fv, fi)

        ov_v[pl.ds(r * L, L)] = fv
        oi_v[pl.ds(r * L, L)] = fi
        return nbv, nbi

    lax.fori_loop(0, qw, one_query, phase2(0))
    pltpu.sync_copy(ov_v, vals_hbm.at[pl.ds(base * L, qw * L)])
    pltpu.sync_copy(oi_v, idx_hbm.at[pl.ds(base * L, qw * L)])


def _topk_sc(m1_flat, scores_rows, qn):
    import functools
    qw = qn // NW
    kern = pl.kernel(
        functools.partial(_topk_sc_body, qn),
        out_type=[
            jax.ShapeDtypeStruct((qn * L,), jnp.float32),
            jax.ShapeDtypeStruct((qn * L,), jnp.int32),
        ],
        mesh=plsc.VectorSubcoreMesh(core_axis_name="c", subcore_axis_name="s"),
        compiler_params=pltpu.CompilerParams(needs_layout_passes=False),
        scratch_types=[
            pltpu.VMEM((qw * NB1P,), jnp.float32),
            pltpu.VMEM((L,), jnp.int32),
            pltpu.VMEM((L, G1), jnp.float32),
            pltpu.VMEM((qw * L,), jnp.float32),
            pltpu.VMEM((qw * L,), jnp.int32),
            pltpu.SemaphoreType.DMA,
        ],
    )
    return kern(m1_flat, scores_rows)


def kernel(queries, keys, k):
    pad = jnp.zeros((KPAD - K, D), keys.dtype).at[:, 0].set(1e18)
    tail = jnp.concatenate([keys[(NTILES - 1) * KT:], pad], axis=0)
    nsplit = 1
    qh = Q // nsplit
    vparts, iparts = [], []
    for part in range(nsplit):
        qs = queries[part * qh:(part + 1) * qh]
        scores, m1q = _phase1(qs, keys, tail)
        vf, i_f = _topk_sc(
            m1q.reshape(-1), scores.reshape(NB1 * qh, G1), qh)
        vparts.append(vf.reshape(qh, L))
        iparts.append(i_f.reshape(qh, L))
    qsq = jnp.sum(queries * queries, axis=1, keepdims=True)
    vals = jnp.concatenate(vparts, axis=0) - qsq
    idx = jnp.concatenate(iparts, axis=0)
    idx = idx + jnp.asarray(k - k, idx.dtype)
    return vals, idx


# speedup vs baseline: 1.3008x; 1.0000x over previous
import jax
import jax.numpy as jnp
from jax import lax
from jax.experimental import pallas as pl
from jax.experimental.pallas import tpu as pltpu
from jax.experimental.pallas import tpu_sc as plsc

Q = 1024
D = 128
K = 100000
KT = 4096
NTILES = 25
KPAD = NTILES * KT
G1 = 128
G2 = 1024
NB1 = KPAD // G1
NB1P = 896
NB2 = KPAD // G2
NB2P = 128
BT = KT // G1
NEG = -3.0e38

NC = 2
NS = 16
L = 16
NW = NC * NS
QW = Q // NW


def _phase1_body(q_ref, k_ref, t_ref, s_ref, m1_ref):
    j = pl.program_id(0)
    q2 = q_ref[...] * 2.0
    ones_row = jnp.ones((1, D), jnp.float32)

    def work(kt_ref):
        rows = []
        for sub in range(KT // 256):
            kk = kt_ref[pl.ds(sub * 256, 256), :]
            dot2 = lax.dot_general(
                q2, kk, (((1,), (1,)), ((), ())),
                preferred_element_type=jnp.float32,
                precision=lax.Precision.DEFAULT,
            )
            ksq = lax.dot_general(
                ones_row, kk * kk, (((1,), (1,)), ((), ())),
                preferred_element_type=jnp.float32,
                precision=lax.Precision.HIGHEST,
            )
            s = dot2 - ksq
            for h in range(2):
                lb = sub * 2 + h
                sh = s[:, h * G1:(h + 1) * G1]
                s_ref[lb] = sh
                rows.append(jnp.max(sh, axis=1, keepdims=True))
        m1t = jnp.concatenate(rows, axis=1)
        for c in range(128 // BT):
            @pl.when(j % (128 // BT) == c)
            def _(c=c):
                m1_ref[:, c * BT:(c + 1) * BT] = m1t

    @pl.when(j < NTILES - 1)
    def _():
        work(k_ref)

    @pl.when(j == NTILES - 1)
    def _():
        work(t_ref)


def _phase1(queries, keys, tail):
    qn = queries.shape[0]
    return pl.pallas_call(
        _phase1_body,
        grid=(NTILES,),
        in_specs=[
            pl.BlockSpec((qn, D), lambda j: (0, 0)),
            pl.BlockSpec((KT, D), lambda j: (jnp.minimum(j, NTILES - 2), 0)),
            pl.BlockSpec((KT, D), lambda j: (0, 0)),
        ],
        out_specs=[
            pl.BlockSpec((BT, qn, G1), lambda j: (j, 0, 0)),
            pl.BlockSpec((qn, 128), lambda j: (0, j // (128 // BT))),
        ],
        out_shape=[
            jax.ShapeDtypeStruct((NB1, qn, G1), jnp.float32),
            jax.ShapeDtypeStruct((qn, NB1P), jnp.float32),
        ],
    )(queries, keys, tail)


_GATHER_DNUMS = lax.GatherDimensionNumbers(
    offset_dims=(), collapsed_slice_dims=(0,), start_index_map=(0,))


def _permute(x, perm):
    return lax.gather(x, perm[:, None], _GATHER_DNUMS, slice_sizes=(1,),
                      mode=lax.GatherScatterMode.PROMISE_IN_BOUNDS)


def _merge16(av, ai, bv, bi):
    rbv = lax.rev(bv, (0,))
    rbi = lax.rev(bi, (0,))
    take = (av > rbv) | ((av == rbv) & (ai < rbi))
    cv = jnp.where(take, av, rbv)
    ci = jnp.where(take, ai, rbi)
    cv, ci = plsc.sort_key_val(cv, ci, descending=True)
    return cv, ci


def _merge_tree(pairs):
    while len(pairs) > 1:
        nxt = []
        for a in range(0, len(pairs) - 1, 2):
            (av, ai), (bv, bi) = pairs[a], pairs[a + 1]
            nxt.append(_merge16(av, ai, bv, bi))
        if len(pairs) % 2:
            nxt.append(pairs[-1])
        pairs = nxt
    return pairs[0]


def _tie_fix(v, i):
    lane = lax.iota(jnp.int32, L)
    for parity in (0, 1, 0, 1):
        partner = lane - 2 * ((lane - parity) % 2) + 1
        partner = jnp.clip(partner, 0, L - 1)
        pv = _permute(v, partner)
        pi = _permute(i, partner)
        win = (v > pv) | ((v == pv) & (i < pi)) | (partner == lane)
        first = partner > lane
        keep_self = jnp.where(first, win, ~win)
        v = jnp.where(keep_self, v, pv)
        i = jnp.where(keep_self, i, pi)
    return v, i


def _topk_sc_body(qn, m1_hbm, sc_hbm, vals_hbm, idx_hbm,
                  m1_v, gidx_v, gath_v, ov_v, oi_v, sem):
    qw = qn // NW
    wid = lax.axis_index("c") * NS + lax.axis_index("s")
    base = wid * qw
    pltpu.sync_copy(m1_hbm.at[pl.ds(base * NB1P, qw * NB1P)], m1_v)

    lane = lax.iota(jnp.int32, L)

    def phase2(r):
        pairs = []
        for c in range(NB1 // L):
            v = m1_v[pl.ds(r * NB1P + c * L, L)]
            i = lane + c * L
            pairs.append(plsc.sort_key_val(v, i, descending=True))
        return _merge_tree(pairs)

    def one_query(r, carry):
        q = base + r
        bv, bi = carry
        gidx_v[...] = bi * qn + q
        cp = pltpu.async_copy(sc_hbm.at[gidx_v], gath_v, sem)
        nbv, nbi = phase2(jnp.minimum(r + 1, qw - 1))
        cp.wait()

        pairs = []
        for g in range(L * (G1 // L) // L):
            segid = lane + g * L
            rowv = segid >> 3
            colbase = (segid & 7) * L
            m = plsc.load_gather(gath_v, [rowv, colbase])
            for off in range(1, L):
                m = jnp.maximum(
                    m, plsc.load_gather(gath_v, [rowv, colbase + off]))
            pairs.append(plsc.sort_key_val(m, segid, descending=True))
        sgv, sgi = _merge_tree(pairs)

        pairs = []
        for mth in range(L):
            s = _permute(sgi, jnp.full((L,), mth, jnp.int32))
            blkv = s >> 3
            b = _permute(bi, blkv)
            v = plsc.load_gather(gath_v, [blkv, (s & 7) * L + lane])
            i = b * G1 + (s & 7) * L + lane
            pairs.append(plsc.sort_key_val(v, i, descending=True))
        fv, fi = _merge_tree(pairs)
        fv, fi = _tie_fix(fv, fi)

        ov_v[pl.ds(r * L, L)] = fv
        oi_v[pl.ds(r * L, L)] = fi
        return nbv, nbi

    lax.fori_loop(0, qw, one_query, phase2(0))
    pltpu.sync_copy(ov_v, vals_hbm.at[pl.ds(base * L, qw * L)])
    pltpu.sync_copy(oi_v, idx_hbm.at[pl.ds(base * L, qw * L)])


def _topk_sc(m1_flat, scores_rows, qn):
    import functools
    qw = qn // NW
    kern = pl.kernel(
        functools.partial(_topk_sc_body, qn),
        out_type=[
            jax.ShapeDtypeStruct((qn * L,), jnp.float32),
            jax.ShapeDtypeStruct((qn * L,), jnp.int32),
        ],
        mesh=plsc.VectorSubcoreMesh(core_axis_name="c", subcore_axis_name="s"),
        compiler_params=pltpu.CompilerParams(needs_layout_passes=False),
        scratch_types=[
            pltpu.VMEM((qw * NB1P,), jnp.float32),
            pltpu.VMEM((L,), jnp.int32),
            pltpu.VMEM((L, G1), jnp.float32),
            pltpu.VMEM((qw * L,), jnp.float32),
            pltpu.VMEM((qw * L,), jnp.int32),
            pltpu.SemaphoreType.DMA,
        ],
    )
    return kern(m1_flat, scores_rows)


def kernel(queries, keys, k):
    pad = jnp.zeros((KPAD - K, D), keys.dtype).at[:, 0].set(1e18)
    tail = jnp.concatenate([keys[(NTILES - 1) * KT:], pad], axis=0)
    nsplit = 1
    qh = Q // nsplit
    vparts, iparts = [], []
    for part in range(nsplit):
        qs = queries[part * qh:(part + 1) * qh]
        scores, m1q = _phase1(qs, keys, tail)
        vf, i_f = _topk_sc(
            m1q.reshape(-1), scores.reshape(NB1 * qh, G1), qh)
        vparts.append(vf.reshape(qh, L))
        iparts.append(i_f.reshape(qh, L))
    qsq = jnp.sum(queries * queries, axis=1, keepdims=True)
    vals = jnp.concatenate(vparts, axis=0) - qsq
    idx = jnp.concatenate(iparts, axis=0)
    idx = idx + jnp.asarray(k - k, idx.dtype)
    return vals, idx


# speedup vs baseline: 1.3040x; 1.0024x over previous
import jax
import jax.numpy as jnp
from jax import lax
from jax.experimental import pallas as pl
from jax.experimental.pallas import tpu as pltpu
from jax.experimental.pallas import tpu_sc as plsc

Q = 1024
D = 128
K = 100000
KT = 4096
NTILES = 25
KPAD = NTILES * KT
G1 = 128
NB1 = KPAD // G1
NB1P = 896
BT = KT // G1
NEG = -3.0e38

NC = 2
NS = 16
L = 16
NW = NC * NS
QW = Q // NW


def _phase1_body(q_ref, k_ref, t_ref, s_ref, m1_ref):
    j = pl.program_id(0)
    q2 = q_ref[...] * 2.0
    ones_row = jnp.ones((1, D), jnp.float32)

    def work(kt_ref):
        rows = []
        for sub in range(KT // 256):
            kk = kt_ref[pl.ds(sub * 256, 256), :]
            dot2 = lax.dot_general(
                q2, kk, (((1,), (1,)), ((), ())),
                preferred_element_type=jnp.float32,
                precision=lax.Precision.DEFAULT,
            )
            ksq = lax.dot_general(
                ones_row, kk * kk, (((1,), (1,)), ((), ())),
                preferred_element_type=jnp.float32,
                precision=lax.Precision.HIGHEST,
            )
            s = dot2 - ksq
            for h in range(2):
                lb = sub * 2 + h
                sh = s[:, h * G1:(h + 1) * G1]
                s_ref[lb] = sh
                rows.append(jnp.max(sh, axis=1, keepdims=True))
        m1t = jnp.concatenate(rows, axis=1)
        for c in range(128 // BT):
            @pl.when(j % (128 // BT) == c)
            def _(c=c):
                m1_ref[:, c * BT:(c + 1) * BT] = m1t

    @pl.when(j < NTILES - 1)
    def _():
        work(k_ref)

    @pl.when(j == NTILES - 1)
    def _():
        work(t_ref)


def _phase1(queries, keys, tail):
    qn = queries.shape[0]
    return pl.pallas_call(
        _phase1_body,
        grid=(NTILES,),
        in_specs=[
            pl.BlockSpec((qn, D), lambda j: (0, 0)),
            pl.BlockSpec((KT, D), lambda j: (jnp.minimum(j, NTILES - 2), 0)),
            pl.BlockSpec((KT, D), lambda j: (0, 0)),
        ],
        out_specs=[
            pl.BlockSpec((BT, qn, G1), lambda j: (j, 0, 0)),
            pl.BlockSpec((qn, 128), lambda j: (0, j // (128 // BT))),
        ],
        out_shape=[
            jax.ShapeDtypeStruct((NB1, qn, G1), jnp.float32),
            jax.ShapeDtypeStruct((qn, NB1P), jnp.float32),
        ],
    )(queries, keys, tail)


_GATHER_DNUMS = lax.GatherDimensionNumbers(
    offset_dims=(), collapsed_slice_dims=(0,), start_index_map=(0,))


def _permute(x, perm):
    return lax.gather(x, perm[:, None], _GATHER_DNUMS, slice_sizes=(1,),
                      mode=lax.GatherScatterMode.PROMISE_IN_BOUNDS)


def _merge16(av, ai, bv, bi):
    rbv = lax.rev(bv, (0,))
    rbi = lax.rev(bi, (0,))
    take = (av > rbv) | ((av == rbv) & (ai < rbi))
    cv = jnp.where(take, av, rbv)
    ci = jnp.where(take, ai, rbi)
    cv, ci = plsc.sort_key_val(cv, ci, descending=True)
    return cv, ci


def _merge_tree(pairs):
    while len(pairs) > 1:
        nxt = []
        for a in range(0, len(pairs) - 1, 2):
            (av, ai), (bv, bi) = pairs[a], pairs[a + 1]
            nxt.append(_merge16(av, ai, bv, bi))
        if len(pairs) % 2:
            nxt.append(pairs[-1])
        pairs = nxt
    return pairs[0]


def _tie_fix(v, i):
    lane = lax.iota(jnp.int32, L)
    for parity in (0, 1, 0, 1):
        partner = lane - 2 * ((lane - parity) % 2) + 1
        partner = jnp.clip(partner, 0, L - 1)
        pv = _permute(v, partner)
        pi = _permute(i, partner)
        win = (v > pv) | ((v == pv) & (i < pi)) | (partner == lane)
        first = partner > lane
        keep_self = jnp.where(first, win, ~win)
        v = jnp.where(keep_self, v, pv)
        i = jnp.where(keep_self, i, pi)
    return v, i


def _topk_sc_body(qn, m1_hbm, sc_hbm, vals_hbm, idx_hbm,
                  m1_v, gidx_v, gath_v, ov_v, oi_v, sem):
    qw = qn // NW
    wid = lax.axis_index("c") * NS + lax.axis_index("s")
    base = wid * qw
    pltpu.sync_copy(m1_hbm.at[pl.ds(base * NB1P, qw * NB1P)], m1_v)

    lane = lax.iota(jnp.int32, L)

    def phase2(r):
        pairs = []
        for c in range(NB1 // L):
            v = m1_v[pl.ds(r * NB1P + c * L, L)]
            i = lane + c * L
            pairs.append(plsc.sort_key_val(v, i, descending=True))
        return _merge_tree(pairs)

    def one_query(r, carry):
        q = base + r
        bv, bi = carry
        gidx_v[...] = bi * qn + q
        cp = pltpu.async_copy(sc_hbm.at[gidx_v], gath_v, sem)
        nbv, nbi = phase2(jnp.minimum(r + 1, qw - 1))
        cp.wait()

        pairs = []
        for g in range(L * (G1 // L) // L):
            segid = lane + g * L
            rowv = segid >> 3
            colbase = (segid & 7) * L
            m = plsc.load_gather(gath_v, [rowv, colbase])
            for off in range(1, L):
                m = jnp.maximum(
                    m, plsc.load_gather(gath_v, [rowv, colbase + off]))
            pairs.append(plsc.sort_key_val(m, segid, descending=True))
        sgv, sgi = _merge_tree(pairs)

        pairs = []
        for mth in range(L):
            s = _permute(sgi, jnp.full((L,), mth, jnp.int32))
            blkv = s >> 3
            b = _permute(bi, blkv)
            v = plsc.load_gather(gath_v, [blkv, (s & 7) * L + lane])
            i = b * G1 + (s & 7) * L + lane
            pairs.append(plsc.sort_key_val(v, i, descending=True))
        fv, fi = _merge_tree(pairs)
        fv, fi = _tie_fix(fv, fi)

        ov_v[pl.ds(r * L, L)] = fv
        oi_v[pl.ds(r * L, L)] = fi
        return nbv, nbi

    lax.fori_loop(0, qw, one_query, phase2(0))
    pltpu.sync_copy(ov_v, vals_hbm.at[pl.ds(base * L, qw * L)])
    pltpu.sync_copy(oi_v, idx_hbm.at[pl.ds(base * L, qw * L)])


def _topk_sc(m1_flat, scores_rows, qn):
    import functools
    qw = qn // NW
    kern = pl.kernel(
        functools.partial(_topk_sc_body, qn),
        out_type=[
            jax.ShapeDtypeStruct((qn * L,), jnp.float32),
            jax.ShapeDtypeStruct((qn * L,), jnp.int32),
        ],
        mesh=plsc.VectorSubcoreMesh(core_axis_name="c", subcore_axis_name="s"),
        compiler_params=pltpu.CompilerParams(needs_layout_passes=False),
        scratch_types=[
            pltpu.VMEM((qw * NB1P,), jnp.float32),
            pltpu.VMEM((L,), jnp.int32),
            pltpu.VMEM((L, G1), jnp.float32),
            pltpu.VMEM((qw * L,), jnp.float32),
            pltpu.VMEM((qw * L,), jnp.int32),
            pltpu.SemaphoreType.DMA,
        ],
    )
    return kern(m1_flat, scores_rows)


def kernel(queries, keys, k):
    pad = jnp.zeros((KPAD - K, D), keys.dtype).at[:, 0].set(1e18)
    tail = jnp.concatenate([keys[(NTILES - 1) * KT:], pad], axis=0)
    scores, m1q = _phase1(queries, keys, tail)
    vals_flat, idx_flat = _topk_sc(
        m1q.reshape(-1), scores.reshape(NB1 * Q, G1), Q)
    qsq = jnp.sum(queries * queries, axis=1, keepdims=True)
    vals = vals_flat.reshape(Q, L) - qsq
    idx = idx_flat.reshape(Q, L)
    idx = idx + jnp.asarray(k - k, idx.dtype)
    return vals, idx
